# baseline (speedup 1.0000x reference)
import functools

import jax
import jax.numpy as jnp
from jax import lax
from jax.experimental import pallas as pl
from jax.experimental.pallas import tpu as pltpu
from jax.experimental.pallas import tpu_sc as plsc

B = 4096
IN = 512
H1 = 512
H2 = 256
C = 16

NC = 2
NS = 16
L = 16
NW = NC * NS
ROWS_PER_W = B // NW
PASS_B = 64
NCHUNK = B // PASS_B

_SPLAT_DNUMS = lax.GatherDimensionNumbers(
    offset_dims=(), collapsed_slice_dims=(0,), start_index_map=(0,))


def _splat_lane(vec, k):
    idx = jnp.full((L, 1), k, jnp.int32)
    return lax.gather(vec, idx, _SPLAT_DNUMS, slice_sizes=(1,),
                      mode=lax.GatherScatterMode.PROMISE_IN_BOUNDS)


def _layer_loop(in_buf, in_words, conn_buf, op_buf, out_buf, out_f, iota):
    def body(j, carry):
        crow = conn_buf[j, :]
        opv = op_buf[j, :]
        jsp = jnp.full((L,), 0, jnp.int32) + j * PASS_B
        idx = [_splat_lane(crow, k) + iota for k in range(C)]
        for bc in range(PASS_B // L):
            base = bc * L
            view = in_buf.at[pl.ds(base, in_words - base)]
            vals = [plsc.load_gather(view, [idx[k]]) for k in range(C)]
            mns = vals
            mxs = vals
            while len(mns) > 1:
                mns = [jnp.minimum(a, b) for a, b in zip(mns[::2], mns[1::2])]
                mxs = [jnp.maximum(a, b) for a, b in zip(mxs[::2], mxs[1::2])]
            res = jnp.where(opv == 1, mxs[0], mns[0])
            plsc.store_scatter(out_buf, [jsp + (iota + base)], res)
        return carry

    lax.fori_loop(0, out_f, body, 0)


def _make_kernel():
    mesh = plsc.VectorSubcoreMesh(core_axis_name="c", subcore_axis_name="s")

    @functools.partial(
        pl.kernel,
        mesh=mesh,
        out_type=jax.ShapeDtypeStruct((NCHUNK, IN * PASS_B), jnp.float32),
        compiler_params=pltpu.CompilerParams(
            use_tc_tiling_on_sc=False, needs_layout_passes=False),
        scratch_types=[
            pltpu.VMEM((IN * PASS_B,), jnp.float32),
            pltpu.VMEM((H1 * PASS_B,), jnp.float32),
            pltpu.VMEM((H2 * PASS_B,), jnp.float32),
            pltpu.VMEM((H1, C), jnp.int32),
            pltpu.VMEM((H2, C), jnp.int32),
            pltpu.VMEM((IN, C), jnp.int32),
            pltpu.VMEM((H1, C), jnp.int32),
            pltpu.VMEM((H2, C), jnp.int32),
            pltpu.VMEM((IN, C), jnp.int32),
        ],
    )
    def k(xt_hbm, c1_hbm, c2_hbm, co_hbm, o1_hbm, o2_hbm, oo_hbm, out_hbm,
          xbuf, h1buf, h2buf, c1b, c2b, cob, o1b, o2b, oob):
        wid = lax.axis_index("s") * NC + lax.axis_index("c")
        pltpu.sync_copy(c1_hbm, c1b)
        pltpu.sync_copy(c2_hbm, c2b)
        pltpu.sync_copy(co_hbm, cob)
        pltpu.sync_copy(o1_hbm, o1b)
        pltpu.sync_copy(o2_hbm, o2b)
        pltpu.sync_copy(oo_hbm, oob)
        iota = lax.iota(jnp.int32, L)
        for p in range(ROWS_PER_W // PASS_B):
            chunk = wid * (ROWS_PER_W // PASS_B) + p
            pltpu.sync_copy(xt_hbm.at[chunk], xbuf)
            _layer_loop(xbuf, IN * PASS_B, c1b, o1b, h1buf, H1, iota)
            _layer_loop(h1buf, H1 * PASS_B, c2b, o2b, h2buf, H2, iota)
            _layer_loop(h2buf, H2 * PASS_B, cob, oob, xbuf, IN, iota)
            pltpu.sync_copy(xbuf, out_hbm.at[chunk])

    return k


@jax.jit
def kernel(x, conn1, conn2, conn_out, op1, op2, op_out):
    o1 = jnp.broadcast_to(op1[:, None], (H1, C))
    o2 = jnp.broadcast_to(op2[:, None], (H2, C))
    oo = jnp.broadcast_to(op_out[:, None], (IN, C))
    xt = x.reshape(NCHUNK, PASS_B, IN).transpose(0, 2, 1).reshape(
        NCHUNK, IN * PASS_B)
    outt = _make_kernel()(xt, conn1 * PASS_B, conn2 * PASS_B,
                          conn_out * PASS_B, o1, o2, oo)
    return outt.reshape(NCHUNK, IN, PASS_B).transpose(0, 2, 1).reshape(B, IN)


# speedup vs baseline: 1.1742x; 1.1742x over previous
import functools

import jax
import jax.numpy as jnp
from jax import lax
from jax.experimental import pallas as pl
from jax.experimental.pallas import tpu as pltpu
from jax.experimental.pallas import tpu_sc as plsc

B = 4096
IN = 512
H1 = 512
H2 = 256
C = 16

NC = 2
NS = 16
L = 16
NW = NC * NS
ROWS_PER_W = B // NW
PASS_B = 64
NCHUNK = B // PASS_B

_SPLAT_DNUMS = lax.GatherDimensionNumbers(
    offset_dims=(), collapsed_slice_dims=(0,), start_index_map=(0,))


def _splat_lane(vec, k):
    idx = jnp.full((L, 1), k, jnp.int32)
    return lax.gather(vec, idx, _SPLAT_DNUMS, slice_sizes=(1,),
                      mode=lax.GatherScatterMode.PROMISE_IN_BOUNDS)


def _layer_loop(in_buf, in_words, conn_buf, op_buf, out_buf, out_f, iota):
    def body(j, carry):
        crow = conn_buf[j, :]
        opv = op_buf[j, :]
        jsp = jnp.full((L,), 0, jnp.int32) + j * PASS_B
        idx = [_splat_lane(crow, k) + iota for k in range(C)]
        for bc in range(PASS_B // L):
            base = bc * L
            view = in_buf.at[pl.ds(base, in_words - base)]
            h = C // 2
            mna = plsc.load_gather(view, [idx[0]])
            mxa = mna
            mnb = plsc.load_gather(view, [idx[h]])
            mxb = mnb
            for k in range(1, h):
                va = plsc.load_gather(view, [idx[k]])
                vb = plsc.load_gather(view, [idx[h + k]])
                mna = jnp.minimum(mna, va)
                mxa = jnp.maximum(mxa, va)
                mnb = jnp.minimum(mnb, vb)
                mxb = jnp.maximum(mxb, vb)
            res = jnp.where(opv == 1, jnp.maximum(mxa, mxb),
                            jnp.minimum(mna, mnb))
            plsc.store_scatter(out_buf, [jsp + (iota + base)], res)
        return carry

    lax.fori_loop(0, out_f, body, 0)


def _make_kernel():
    mesh = plsc.VectorSubcoreMesh(core_axis_name="c", subcore_axis_name="s")

    @functools.partial(
        pl.kernel,
        mesh=mesh,
        out_type=jax.ShapeDtypeStruct((NCHUNK, IN * PASS_B), jnp.float32),
        compiler_params=pltpu.CompilerParams(
            use_tc_tiling_on_sc=False, needs_layout_passes=False),
        scratch_types=[
            pltpu.VMEM((IN * PASS_B,), jnp.float32),
            pltpu.VMEM((H1 * PASS_B,), jnp.float32),
            pltpu.VMEM((H2 * PASS_B,), jnp.float32),
            pltpu.VMEM((H1, C), jnp.int32),
            pltpu.VMEM((H2, C), jnp.int32),
            pltpu.VMEM((IN, C), jnp.int32),
            pltpu.VMEM((H1, C), jnp.int32),
            pltpu.VMEM((H2, C), jnp.int32),
            pltpu.VMEM((IN, C), jnp.int32),
        ],
    )
    def k(xt_hbm, c1_hbm, c2_hbm, co_hbm, o1_hbm, o2_hbm, oo_hbm, out_hbm,
          xbuf, h1buf, h2buf, c1b, c2b, cob, o1b, o2b, oob):
        wid = lax.axis_index("s") * NC + lax.axis_index("c")
        pltpu.sync_copy(c1_hbm, c1b)
        pltpu.sync_copy(c2_hbm, c2b)
        pltpu.sync_copy(co_hbm, cob)
        pltpu.sync_copy(o1_hbm, o1b)
        pltpu.sync_copy(o2_hbm, o2b)
        pltpu.sync_copy(oo_hbm, oob)
        iota = lax.iota(jnp.int32, L)
        for p in range(ROWS_PER_W // PASS_B):
            chunk = wid * (ROWS_PER_W // PASS_B) + p
            pltpu.sync_copy(xt_hbm.at[chunk], xbuf)
            _layer_loop(xbuf, IN * PASS_B, c1b, o1b, h1buf, H1, iota)
            _layer_loop(h1buf, H1 * PASS_B, c2b, o2b, h2buf, H2, iota)
            _layer_loop(h2buf, H2 * PASS_B, cob, oob, xbuf, IN, iota)
            pltpu.sync_copy(xbuf, out_hbm.at[chunk])

    return k


@jax.jit
def kernel(x, conn1, conn2, conn_out, op1, op2, op_out):
    o1 = jnp.broadcast_to(op1[:, None], (H1, C))
    o2 = jnp.broadcast_to(op2[:, None], (H2, C))
    oo = jnp.broadcast_to(op_out[:, None], (IN, C))
    xt = x.reshape(NCHUNK, PASS_B, IN).transpose(0, 2, 1).reshape(
        NCHUNK, IN * PASS_B)
    outt = _make_kernel()(xt, conn1 * PASS_B, conn2 * PASS_B,
                          conn_out * PASS_B, o1, o2, oo)
    return outt.reshape(NCHUNK, IN, PASS_B).transpose(0, 2, 1).reshape(B, IN)


# speedup vs baseline: 1.3955x; 1.1885x over previous
import functools

import jax
import jax.numpy as jnp
from jax import lax
from jax.experimental import pallas as pl
from jax.experimental.pallas import tpu as pltpu
from jax.experimental.pallas import tpu_sc as plsc

B = 4096
IN = 512
H1 = 512
H2 = 256
C = 16

NC = 2
NS = 16
L = 16
NW = NC * NS
ROWS_PER_W = B // NW
W_PER_F = ROWS_PER_W // 2

_SPLAT_DNUMS = lax.GatherDimensionNumbers(
    offset_dims=(), collapsed_slice_dims=(0,), start_index_map=(0,))


def _splat_lane(vec, k):
    idx = jnp.full((L, 1), k, jnp.int32)
    return lax.gather(vec, idx, _SPLAT_DNUMS, slice_sizes=(1,),
                      mode=lax.GatherScatterMode.PROMISE_IN_BOUNDS)


def _layer_loop(in_buf, in_words, conn_buf, op_buf, out_buf, out_f, iota):
    def body(j, carry):
        crow = conn_buf[j, :]
        opv = op_buf[j, :]
        jsp = jnp.full((L,), 0, jnp.int32) + j * W_PER_F
        idx = [_splat_lane(crow, k) + iota for k in range(C)]
        for bc in range(W_PER_F // L):
            base = bc * L
            view = in_buf.at[pl.ds(base, in_words - base)]
            mn = plsc.bitcast(plsc.load_gather(view, [idx[0]]), jnp.bfloat16)
            mx = mn
            for k in range(1, C):
                v = plsc.bitcast(plsc.load_gather(view, [idx[k]]),
                                 jnp.bfloat16)
                mn = jnp.minimum(mn, v)
                mx = jnp.maximum(mx, v)
            res = jnp.where(opv == 1, mx, mn)
            plsc.store_scatter(out_buf, [jsp + (iota + base)],
                               plsc.bitcast(res, jnp.int32))
        return carry

    lax.fori_loop(0, out_f, body, 0)


def _make_kernel():
    mesh = plsc.VectorSubcoreMesh(core_axis_name="c", subcore_axis_name="s")

    @functools.partial(
        pl.kernel,
        mesh=mesh,
        out_type=jax.ShapeDtypeStruct((NW, IN * W_PER_F), jnp.int32),
        compiler_params=pltpu.CompilerParams(
            use_tc_tiling_on_sc=False, needs_layout_passes=False),
        scratch_types=[
            pltpu.VMEM((IN * W_PER_F,), jnp.int32),
            pltpu.VMEM((H1 * W_PER_F,), jnp.int32),
            pltpu.VMEM((H2 * W_PER_F,), jnp.int32),
            pltpu.VMEM((H1, C), jnp.int32),
            pltpu.VMEM((H2, C), jnp.int32),
            pltpu.VMEM((IN, C), jnp.int32),
            pltpu.VMEM((H1, 2 * L), jnp.int16),
            pltpu.VMEM((H2, 2 * L), jnp.int16),
            pltpu.VMEM((IN, 2 * L), jnp.int16),
        ],
    )
    def k(xp_hbm, c1_hbm, c2_hbm, co_hbm, o1_hbm, o2_hbm, oo_hbm, out_hbm,
          xbuf, h1buf, h2buf, c1b, c2b, cob, o1b, o2b, oob):
        wid = lax.axis_index("s") * NC + lax.axis_index("c")
        pltpu.sync_copy(c1_hbm, c1b)
        pltpu.sync_copy(c2_hbm, c2b)
        pltpu.sync_copy(co_hbm, cob)
        pltpu.sync_copy(o1_hbm, o1b)
        pltpu.sync_copy(o2_hbm, o2b)
        pltpu.sync_copy(oo_hbm, oob)
        iota = lax.iota(jnp.int32, L)
        pltpu.sync_copy(xp_hbm.at[wid], xbuf)
        _layer_loop(xbuf, IN * W_PER_F, c1b, o1b, h1buf, H1, iota)
        _layer_loop(h1buf, H1 * W_PER_F, c2b, o2b, h2buf, H2, iota)
        _layer_loop(h2buf, H2 * W_PER_F, cob, oob, xbuf, IN, iota)
        pltpu.sync_copy(xbuf, out_hbm.at[wid])

    return k


@jax.jit
def kernel(x, conn1, conn2, conn_out, op1, op2, op_out):
    o1 = jnp.broadcast_to(op1[:, None], (H1, 2 * L)).astype(jnp.int16)
    o2 = jnp.broadcast_to(op2[:, None], (H2, 2 * L)).astype(jnp.int16)
    oo = jnp.broadcast_to(op_out[:, None], (IN, 2 * L)).astype(jnp.int16)
    xbf = x.astype(jnp.bfloat16)
    xp = lax.bitcast_convert_type(
        xbf.reshape(NW, W_PER_F, 2, IN).transpose(0, 3, 1, 2), jnp.int32
    ).reshape(NW, IN * W_PER_F)
    outp = _make_kernel()(xp, conn1 * W_PER_F, conn2 * W_PER_F,
                          conn_out * W_PER_F, o1, o2, oo)
    outbf = lax.bitcast_convert_type(
        outp.reshape(NW, IN, W_PER_F), jnp.bfloat16)
    return outbf.transpose(0, 2, 3, 1).reshape(B, IN).astype(jnp.float32)


# speedup vs baseline: 1.4083x; 1.0092x over previous
import functools

import jax
import jax.numpy as jnp
from jax import lax
from jax.experimental import pallas as pl
from jax.experimental.pallas import tpu as pltpu
from jax.experimental.pallas import tpu_sc as plsc

B = 4096
IN = 512
H1 = 512
H2 = 256
C = 16

NC = 2
NS = 16
L = 16
NW = NC * NS
ROWS_PER_W = B // NW
W_PER_F = ROWS_PER_W // 2

_SPLAT_DNUMS = lax.GatherDimensionNumbers(
    offset_dims=(), collapsed_slice_dims=(0,), start_index_map=(0,))


def _splat_lane(vec, k):
    idx = jnp.full((L, 1), k, jnp.int32)
    return lax.gather(vec, idx, _SPLAT_DNUMS, slice_sizes=(1,),
                      mode=lax.GatherScatterMode.PROMISE_IN_BOUNDS)


def _layer_loop(in_buf, in_words, conn_buf, op_buf, out_buf, out_f, iota):
    def body(j, carry):
        crow = conn_buf[j, :]
        opv = op_buf[j, :]
        jsp = jnp.full((L,), 0, jnp.int32) + j * W_PER_F
        idx = [_splat_lane(crow, k) + iota for k in range(C)]
        for bc in range(W_PER_F // L):
            base = bc * L
            view = in_buf.at[pl.ds(base, in_words - base)]
            mn = plsc.bitcast(plsc.load_gather(view, [idx[0]]), jnp.bfloat16)
            mx = mn
            for k in range(1, C):
                v = plsc.bitcast(plsc.load_gather(view, [idx[k]]),
                                 jnp.bfloat16)
                mn = jnp.minimum(mn, v)
                mx = jnp.maximum(mx, v)
            res = jnp.where(opv == 1, plsc.bitcast(mx, jnp.int32),
                            plsc.bitcast(mn, jnp.int32))
            plsc.store_scatter(out_buf, [jsp + (iota + base)], res)
        return carry

    lax.fori_loop(0, out_f, body, 0)


def _make_kernel():
    mesh = plsc.VectorSubcoreMesh(core_axis_name="c", subcore_axis_name="s")

    @functools.partial(
        pl.kernel,
        mesh=mesh,
        out_type=jax.ShapeDtypeStruct((NW, IN * W_PER_F), jnp.int32),
        compiler_params=pltpu.CompilerParams(
            use_tc_tiling_on_sc=False, needs_layout_passes=False),
        scratch_types=[
            pltpu.VMEM((IN * W_PER_F,), jnp.int32),
            pltpu.VMEM((H1 * W_PER_F,), jnp.int32),
            pltpu.VMEM((H2 * W_PER_F,), jnp.int32),
            pltpu.VMEM((H1, C), jnp.int32),
            pltpu.VMEM((H2, C), jnp.int32),
            pltpu.VMEM((IN, C), jnp.int32),
            pltpu.VMEM((H1, C), jnp.int32),
            pltpu.VMEM((H2, C), jnp.int32),
            pltpu.VMEM((IN, C), jnp.int32),
        ],
    )
    def k(xp_hbm, c1_hbm, c2_hbm, co_hbm, o1_hbm, o2_hbm, oo_hbm, out_hbm,
          xbuf, h1buf, h2buf, c1b, c2b, cob, o1b, o2b, oob):
        wid = lax.axis_index("s") * NC + lax.axis_index("c")
        pltpu.sync_copy(c1_hbm, c1b)
        pltpu.sync_copy(c2_hbm, c2b)
        pltpu.sync_copy(co_hbm, cob)
        pltpu.sync_copy(o1_hbm, o1b)
        pltpu.sync_copy(o2_hbm, o2b)
        pltpu.sync_copy(oo_hbm, oob)
        iota = lax.iota(jnp.int32, L)
        pltpu.sync_copy(xp_hbm.at[wid], xbuf)
        _layer_loop(xbuf, IN * W_PER_F, c1b, o1b, h1buf, H1, iota)
        _layer_loop(h1buf, H1 * W_PER_F, c2b, o2b, h2buf, H2, iota)
        _layer_loop(h2buf, H2 * W_PER_F, cob, oob, xbuf, IN, iota)
        pltpu.sync_copy(xbuf, out_hbm.at[wid])

    return k


@jax.jit
def kernel(x, conn1, conn2, conn_out, op1, op2, op_out):
    o1 = jnp.broadcast_to(op1[:, None], (H1, C))
    o2 = jnp.broadcast_to(op2[:, None], (H2, C))
    oo = jnp.broadcast_to(op_out[:, None], (IN, C))
    xbf = x.astype(jnp.bfloat16)
    xp = lax.bitcast_convert_type(
        xbf.reshape(NW, W_PER_F, 2, IN).transpose(0, 3, 1, 2), jnp.int32
    ).reshape(NW, IN * W_PER_F)
    outp = _make_kernel()(xp, conn1 * W_PER_F, conn2 * W_PER_F,
                          conn_out * W_PER_F, o1, o2, oo)
    outbf = lax.bitcast_convert_type(
        outp.reshape(NW, IN, W_PER_F), jnp.bfloat16)
    return outbf.transpose(0, 2, 3, 1).reshape(B, IN).astype(jnp.float32)


# speedup vs baseline: 1.9013x; 1.3501x over previous
import functools

import jax
import jax.numpy as jnp
from jax import lax
from jax.experimental import pallas as pl
from jax.experimental.pallas import tpu as pltpu
from jax.experimental.pallas import tpu_sc as plsc

B = 4096
IN = 512
H1 = 512
H2 = 256
C = 16

NC = 2
NS = 16
L = 16
NW = NC * NS
ROWS_PER_W = B // NW
W_PER_F = ROWS_PER_W // 2
PIECE_ROWS = 32
N_PIECES = ROWS_PER_W // PIECE_ROWS

_GATHER_DNUMS = lax.GatherDimensionNumbers(
    offset_dims=(), collapsed_slice_dims=(0,), start_index_map=(0,))


def _lane_gather(vec, idx_vec):
    return lax.gather(vec, idx_vec[:, None], _GATHER_DNUMS, slice_sizes=(1,),
                      mode=lax.GatherScatterMode.PROMISE_IN_BOUNDS)


def _splat_lane(vec, k):
    idx = jnp.full((L,), 0, jnp.int32) + k
    return _lane_gather(vec, idx)


def _transpose16(vs, iota):
    for s in range(4):
        d = 1 << s
        perm = iota ^ d
        bit = (iota & d) != 0
        nvs = list(vs)
        for i in range(16):
            if i & d:
                continue
            j = i | d
            a, b = vs[i], vs[j]
            pa = _lane_gather(a, perm)
            pb = _lane_gather(b, perm)
            nvs[i] = jnp.where(bit, pb, a)
            nvs[j] = jnp.where(bit, b, pa)
        vs = nvs
    return vs


def _layer_loop(in_buf, in_words, conn_buf, op_buf, out_buf, out_f, iota):
    def body(j, carry):
        crow = conn_buf[j, :]
        oprow = op_buf[lax.shift_right_logical(j, 4), :]
        opv = _splat_lane(oprow, lax.bitwise_and(j, 15))
        jsp = jnp.full((L,), 0, jnp.int32) + j * W_PER_F
        idx = [_splat_lane(crow, k) + iota for k in range(C)]
        for bc in range(W_PER_F // L):
            base = bc * L
            view = in_buf.at[pl.ds(base, in_words - base)]
            mn = plsc.bitcast(plsc.load_gather(view, [idx[0]]), jnp.bfloat16)
            mx = mn
            for k in range(1, C):
                v = plsc.bitcast(plsc.load_gather(view, [idx[k]]),
                                 jnp.bfloat16)
                mn = jnp.minimum(mn, v)
                mx = jnp.maximum(mx, v)
            res = jnp.where(opv == 1, plsc.bitcast(mx, jnp.int32),
                            plsc.bitcast(mn, jnp.int32))
            plsc.store_scatter(out_buf, [jsp + (iota + base)], res)
        return carry

    lax.fori_loop(0, out_f, body, 0)


def _make_kernel():
    mesh = plsc.VectorSubcoreMesh(core_axis_name="c", subcore_axis_name="s")

    @functools.partial(
        pl.kernel,
        mesh=mesh,
        out_type=jax.ShapeDtypeStruct((B * IN,), jnp.float32),
        compiler_params=pltpu.CompilerParams(
            use_tc_tiling_on_sc=False, needs_layout_passes=False),
        scratch_types=[
            pltpu.VMEM((PIECE_ROWS * IN,), jnp.float32),
            pltpu.VMEM((IN * W_PER_F,), jnp.int32),
            pltpu.VMEM((H1 * W_PER_F,), jnp.int32),
            pltpu.VMEM((H2 * W_PER_F,), jnp.int32),
            pltpu.VMEM((H1, C), jnp.int32),
            pltpu.VMEM((H2, C), jnp.int32),
            pltpu.VMEM((IN, C), jnp.int32),
            pltpu.VMEM((H1 // L, L), jnp.int32),
            pltpu.VMEM((H2 // L, L), jnp.int32),
            pltpu.VMEM((IN // L, L), jnp.int32),
        ],
    )
    def k(x_hbm, c1_hbm, c2_hbm, co_hbm, o1_hbm, o2_hbm, oo_hbm, out_hbm,
          stage, xp, h1p, h2p, c1b, c2b, cob, o1b, o2b, oob):
        wid = lax.axis_index("s") * NC + lax.axis_index("c")
        pltpu.sync_copy(c1_hbm, c1b)
        pltpu.sync_copy(c2_hbm, c2b)
        pltpu.sync_copy(co_hbm, cob)
        pltpu.sync_copy(o1_hbm, o1b)
        pltpu.sync_copy(o2_hbm, o2b)
        pltpu.sync_copy(oo_hbm, oob)
        iota = lax.iota(jnp.int32, L)

        for p in range(N_PIECES):
            rowbase = (wid * ROWS_PER_W + p * PIECE_ROWS) * IN
            pltpu.sync_copy(x_hbm.at[pl.ds(rowbase, PIECE_ROWS * IN)], stage)
            wbase = p * L

            def pack_body(fb, carry):
                f0 = fb * L
                ws = []
                for wl in range(L):
                    a = stage[pl.ds((2 * wl) * IN + f0, L)]
                    b = stage[pl.ds((2 * wl + 1) * IN + f0, L)]
                    ws.append(plsc.bitcast(plsc.pack(a, b, format=plsc.PackFormat.INTERLEAVED), jnp.int32))
                qs = _transpose16(ws, iota)
                for i in range(L):
                    xp[pl.ds((f0 + i) * W_PER_F + wbase, L)] = qs[i]
                return carry

            lax.fori_loop(0, IN // L, pack_body, 0)

        _layer_loop(xp, IN * W_PER_F, c1b, o1b, h1p, H1, iota)
        _layer_loop(h1p, H1 * W_PER_F, c2b, o2b, h2p, H2, iota)
        _layer_loop(h2p, H2 * W_PER_F, cob, oob, xp, IN, iota)

        for p in range(N_PIECES):
            wbase = p * L

            def unpack_body(fb, carry):
                f0 = fb * L
                vs = [xp[pl.ds((f0 + i) * W_PER_F + wbase, L)]
                      for i in range(L)]
                ws = _transpose16(vs, iota)
                for wl in range(L):
                    a, b = plsc.unpack(plsc.bitcast(ws[wl], jnp.bfloat16), format=plsc.PackFormat.INTERLEAVED)
                    stage[pl.ds((2 * wl) * IN + f0, L)] = a
                    stage[pl.ds((2 * wl + 1) * IN + f0, L)] = b
                return carry

            lax.fori_loop(0, IN // L, unpack_body, 0)
            rowbase = (wid * ROWS_PER_W + p * PIECE_ROWS) * IN
            pltpu.sync_copy(stage, out_hbm.at[pl.ds(rowbase, PIECE_ROWS * IN)])

    return k


@jax.jit
def kernel(x, conn1, conn2, conn_out, op1, op2, op_out):
    outf = _make_kernel()(x.reshape(B * IN), conn1 * W_PER_F,
                          conn2 * W_PER_F, conn_out * W_PER_F,
                          op1.reshape(H1 // L, L), op2.reshape(H2 // L, L),
                          op_out.reshape(IN // L, L))
    return outf.reshape(B, IN)


# speedup vs baseline: 1.9036x; 1.0012x over previous
import functools

import jax
import jax.numpy as jnp
from jax import lax
from jax.experimental import pallas as pl
from jax.experimental.pallas import tpu as pltpu
from jax.experimental.pallas import tpu_sc as plsc

B = 4096
IN = 512
H1 = 512
H2 = 256
C = 16

NC = 2
NS = 16
L = 16
NW = NC * NS
ROWS_PER_W = B // NW
W_PER_F = ROWS_PER_W // 2
PIECE_ROWS = 32
N_PIECES = ROWS_PER_W // PIECE_ROWS

_GATHER_DNUMS = lax.GatherDimensionNumbers(
    offset_dims=(), collapsed_slice_dims=(0,), start_index_map=(0,))


def _lane_gather(vec, idx_vec):
    return lax.gather(vec, idx_vec[:, None], _GATHER_DNUMS, slice_sizes=(1,),
                      mode=lax.GatherScatterMode.PROMISE_IN_BOUNDS)


def _splat_lane(vec, k):
    idx = jnp.full((L,), 0, jnp.int32) + k
    return _lane_gather(vec, idx)


def _transpose16(vs, iota):
    for s in range(4):
        d = 1 << s
        perm = iota ^ d
        bit = (iota & d) != 0
        nvs = list(vs)
        for i in range(16):
            if i & d:
                continue
            j = i | d
            a, b = vs[i], vs[j]
            pa = _lane_gather(a, perm)
            pb = _lane_gather(b, perm)
            nvs[i] = jnp.where(bit, pb, a)
            nvs[j] = jnp.where(bit, b, pa)
        vs = nvs
    return vs


def _layer_loop(in_buf, in_words, conn_buf, op_buf, out_buf, out_f, iota):
    def body(j, carry):
        crow = conn_buf[j, :]
        oprow = op_buf[pl.ds(lax.bitwise_and(j, -16), L)]
        opv = _splat_lane(oprow, lax.bitwise_and(j, 15))
        jsp = jnp.full((L,), 0, jnp.int32) + j * W_PER_F
        idx = [_splat_lane(crow, k) + iota for k in range(C)]
        for bc in range(W_PER_F // L):
            base = bc * L
            view = in_buf.at[pl.ds(base, in_words - base)]
            mn = plsc.bitcast(plsc.load_gather(view, [idx[0]]), jnp.bfloat16)
            mx = mn
            for k in range(1, C):
                v = plsc.bitcast(plsc.load_gather(view, [idx[k]]),
                                 jnp.bfloat16)
                mn = jnp.minimum(mn, v)
                mx = jnp.maximum(mx, v)
            res = jnp.where(opv == 1, plsc.bitcast(mx, jnp.int32),
                            plsc.bitcast(mn, jnp.int32))
            plsc.store_scatter(out_buf, [jsp + (iota + base)], res)
        return carry

    lax.fori_loop(0, out_f, body, 0)


def _make_kernel():
    mesh = plsc.VectorSubcoreMesh(core_axis_name="c", subcore_axis_name="s")

    @functools.partial(
        pl.kernel,
        mesh=mesh,
        out_type=jax.ShapeDtypeStruct((B, IN), jnp.float32),
        compiler_params=pltpu.CompilerParams(
            use_tc_tiling_on_sc=False, needs_layout_passes=False),
        scratch_types=[
            pltpu.VMEM((PIECE_ROWS, IN), jnp.float32),
            pltpu.VMEM((IN * W_PER_F,), jnp.int32),
            pltpu.VMEM((H1 * W_PER_F,), jnp.int32),
            pltpu.VMEM((H2 * W_PER_F,), jnp.int32),
            pltpu.VMEM((H1, C), jnp.int32),
            pltpu.VMEM((H2, C), jnp.int32),
            pltpu.VMEM((IN, C), jnp.int32),
            pltpu.VMEM((H1,), jnp.int32),
            pltpu.VMEM((H2,), jnp.int32),
            pltpu.VMEM((IN,), jnp.int32),
        ],
    )
    def k(x_hbm, c1_hbm, c2_hbm, co_hbm, o1_hbm, o2_hbm, oo_hbm, out_hbm,
          stage, xp, h1p, h2p, c1b, c2b, cob, o1b, o2b, oob):
        wid = lax.axis_index("s") * NC + lax.axis_index("c")
        pltpu.sync_copy(c1_hbm, c1b)
        pltpu.sync_copy(c2_hbm, c2b)
        pltpu.sync_copy(co_hbm, cob)
        pltpu.sync_copy(o1_hbm, o1b)
        pltpu.sync_copy(o2_hbm, o2b)
        pltpu.sync_copy(oo_hbm, oob)
        iota = lax.iota(jnp.int32, L)

        for p in range(N_PIECES):
            rowbase = wid * ROWS_PER_W + p * PIECE_ROWS
            pltpu.sync_copy(x_hbm.at[pl.ds(rowbase, PIECE_ROWS)], stage)
            wbase = p * L

            def pack_body(fb, carry):
                f0 = fb * L
                ws = []
                for wl in range(L):
                    a = stage[2 * wl, pl.ds(f0, L)]
                    b = stage[2 * wl + 1, pl.ds(f0, L)]
                    ws.append(plsc.bitcast(plsc.pack(a, b, format=plsc.PackFormat.INTERLEAVED), jnp.int32))
                qs = _transpose16(ws, iota)
                for i in range(L):
                    xp[pl.ds((f0 + i) * W_PER_F + wbase, L)] = qs[i]
                return carry

            lax.fori_loop(0, IN // L, pack_body, 0)

        _layer_loop(xp, IN * W_PER_F, c1b, o1b, h1p, H1, iota)
        _layer_loop(h1p, H1 * W_PER_F, c2b, o2b, h2p, H2, iota)
        _layer_loop(h2p, H2 * W_PER_F, cob, oob, xp, IN, iota)

        for p in range(N_PIECES):
            wbase = p * L

            def unpack_body(fb, carry):
                f0 = fb * L
                vs = [xp[pl.ds((f0 + i) * W_PER_F + wbase, L)]
                      for i in range(L)]
                ws = _transpose16(vs, iota)
                for wl in range(L):
                    a, b = plsc.unpack(plsc.bitcast(ws[wl], jnp.bfloat16), format=plsc.PackFormat.INTERLEAVED)
                    stage[2 * wl, pl.ds(f0, L)] = a
                    stage[2 * wl + 1, pl.ds(f0, L)] = b
                return carry

            lax.fori_loop(0, IN // L, unpack_body, 0)
            rowbase = wid * ROWS_PER_W + p * PIECE_ROWS
            pltpu.sync_copy(stage, out_hbm.at[pl.ds(rowbase, PIECE_ROWS)])

    return k


@jax.jit
def kernel(x, conn1, conn2, conn_out, op1, op2, op_out):
    return _make_kernel()(x, conn1 * W_PER_F, conn2 * W_PER_F,
                          conn_out * W_PER_F, op1, op2, op_out)


# speedup vs baseline: 1.9269x; 1.0122x over previous
import functools

import jax
import jax.numpy as jnp
from jax import lax
from jax.experimental import pallas as pl
from jax.experimental.pallas import tpu as pltpu
from jax.experimental.pallas import tpu_sc as plsc

B = 4096
IN = 512
H1 = 512
H2 = 256
C = 16

NC = 2
NS = 16
L = 16
NW = NC * NS
ROWS_PER_W = B // NW
W_PER_F = ROWS_PER_W // 2
PIECE_ROWS = 32
N_PIECES = ROWS_PER_W // PIECE_ROWS

_GATHER_DNUMS = lax.GatherDimensionNumbers(
    offset_dims=(), collapsed_slice_dims=(0,), start_index_map=(0,))


def _lane_gather(vec, idx_vec):
    return lax.gather(vec, idx_vec[:, None], _GATHER_DNUMS, slice_sizes=(1,),
                      mode=lax.GatherScatterMode.PROMISE_IN_BOUNDS)


def _splat_lane(vec, k):
    idx = jnp.full((L,), 0, jnp.int32) + k
    return _lane_gather(vec, idx)


def _transpose16(vs, iota):
    for s in range(4):
        d = 1 << s
        perm = iota ^ d
        bit = (iota & d) != 0
        nvs = list(vs)
        for i in range(16):
            if i & d:
                continue
            j = i | d
            a, b = vs[i], vs[j]
            pa = _lane_gather(a, perm)
            pb = _lane_gather(b, perm)
            nvs[i] = jnp.where(bit, pb, a)
            nvs[j] = jnp.where(bit, b, pa)
        vs = nvs
    return vs


def _layer_loop(in_buf, in_words, conn_buf, op_buf, out_buf, out_f, iota):
    def body(j, carry):
        crow = conn_buf[j, :]
        oprow = op_buf[pl.ds(lax.bitwise_and(j, -16), L)]
        opv = _splat_lane(oprow, lax.bitwise_and(j, 15))
        jsp = jnp.full((L,), 0, jnp.int32) + j * W_PER_F
        idx = [_splat_lane(crow, k) + iota for k in range(C)]
        for bc in range(W_PER_F // L):
            base = bc * L
            view = in_buf.at[pl.ds(base, in_words - base)]
            mn = plsc.bitcast(plsc.load_gather(view, [idx[0]]), jnp.bfloat16)
            mx = mn
            for k in range(1, C):
                v = plsc.bitcast(plsc.load_gather(view, [idx[k]]),
                                 jnp.bfloat16)
                mn = jnp.minimum(mn, v)
                mx = jnp.maximum(mx, v)
            res = jnp.where(opv == 1, plsc.bitcast(mx, jnp.int32),
                            plsc.bitcast(mn, jnp.int32))
            plsc.store_scatter(out_buf, [jsp + (iota + base)], res)
        return carry

    lax.fori_loop(0, out_f, body, 0, unroll=2)


def _make_kernel():
    mesh = plsc.VectorSubcoreMesh(core_axis_name="c", subcore_axis_name="s")

    @functools.partial(
        pl.kernel,
        mesh=mesh,
        out_type=jax.ShapeDtypeStruct((B, IN), jnp.float32),
        compiler_params=pltpu.CompilerParams(
            use_tc_tiling_on_sc=False, needs_layout_passes=False),
        scratch_types=[
            pltpu.VMEM((PIECE_ROWS, IN), jnp.float32),
            pltpu.VMEM((IN * W_PER_F,), jnp.int32),
            pltpu.VMEM((H1 * W_PER_F,), jnp.int32),
            pltpu.VMEM((H2 * W_PER_F,), jnp.int32),
            pltpu.VMEM((H1, C), jnp.int32),
            pltpu.VMEM((H2, C), jnp.int32),
            pltpu.VMEM((IN, C), jnp.int32),
            pltpu.VMEM((H1,), jnp.int32),
            pltpu.VMEM((H2,), jnp.int32),
            pltpu.VMEM((IN,), jnp.int32),
        ],
    )
    def k(x_hbm, c1_hbm, c2_hbm, co_hbm, o1_hbm, o2_hbm, oo_hbm, out_hbm,
          stage, xp, h1p, h2p, c1b, c2b, cob, o1b, o2b, oob):
        wid = lax.axis_index("s") * NC + lax.axis_index("c")
        pltpu.sync_copy(c1_hbm, c1b)
        pltpu.sync_copy(c2_hbm, c2b)
        pltpu.sync_copy(co_hbm, cob)
        pltpu.sync_copy(o1_hbm, o1b)
        pltpu.sync_copy(o2_hbm, o2b)
        pltpu.sync_copy(oo_hbm, oob)
        iota = lax.iota(jnp.int32, L)

        for p in range(N_PIECES):
            rowbase = wid * ROWS_PER_W + p * PIECE_ROWS
            pltpu.sync_copy(x_hbm.at[pl.ds(rowbase, PIECE_ROWS)], stage)
            wbase = p * L

            def pack_body(fb, carry):
                f0 = fb * L
                ws = []
                for wl in range(L):
                    a = stage[2 * wl, pl.ds(f0, L)]
                    b = stage[2 * wl + 1, pl.ds(f0, L)]
                    ws.append(plsc.bitcast(plsc.pack(a, b, format=plsc.PackFormat.INTERLEAVED), jnp.int32))
                qs = _transpose16(ws, iota)
                for i in range(L):
                    xp[pl.ds((f0 + i) * W_PER_F + wbase, L)] = qs[i]
                return carry

            lax.fori_loop(0, IN // L, pack_body, 0)

        _layer_loop(xp, IN * W_PER_F, c1b, o1b, h1p, H1, iota)
        _layer_loop(h1p, H1 * W_PER_F, c2b, o2b, h2p, H2, iota)
        _layer_loop(h2p, H2 * W_PER_F, cob, oob, xp, IN, iota)

        for p in range(N_PIECES):
            wbase = p * L

            def unpack_body(fb, carry):
                f0 = fb * L
                vs = [xp[pl.ds((f0 + i) * W_PER_F + wbase, L)]
                      for i in range(L)]
                ws = _transpose16(vs, iota)
                for wl in range(L):
                    a, b = plsc.unpack(plsc.bitcast(ws[wl], jnp.bfloat16), format=plsc.PackFormat.INTERLEAVED)
                    stage[2 * wl, pl.ds(f0, L)] = a
                    stage[2 * wl + 1, pl.ds(f0, L)] = b
                return carry

            lax.fori_loop(0, IN // L, unpack_body, 0)
            rowbase = wid * ROWS_PER_W + p * PIECE_ROWS
            pltpu.sync_copy(stage, out_hbm.at[pl.ds(rowbase, PIECE_ROWS)])

    return k


@jax.jit
def kernel(x, conn1, conn2, conn_out, op1, op2, op_out):
    return _make_kernel()(x, conn1 * W_PER_F, conn2 * W_PER_F,
                          conn_out * W_PER_F, op1, op2, op_out)
